# baseline (speedup 1.0000x reference)
import functools
import jax
import jax.numpy as jnp
from jax import lax
from jax.experimental import pallas as pl
from jax.experimental.pallas import tpu as pltpu
from jax.experimental.pallas import tpu_sc as plsc

N = 68
NPAD = 80
VOCAB = 101
VPAD = 112
EMB = 32
L = 16
NCH = NPAD // L
VCH = VPAD // L
NEWTON_ITERS = 16

_SEG_MEAN = ((0, 10), (10, 36), (36, N))
_SEG_STD = ((0, 10), (10, 26), (36, N))


def _splat(val, dtype=jnp.int32):
    return jnp.full((L,), val, dtype)


def _sc_body(table_hbm, chars_hbm, w_hbm, b_hbm, embT_hbm, parts_hbm,
             table_v, chars_v, w_v, b_v, t2_v, col_v, tmp_v, part_v,
             sem):
    sid = lax.axis_index("s")
    js = [sid, sid + L]

    cps = [pltpu.async_copy(table_hbm, table_v, sem),
           pltpu.async_copy(chars_hbm, chars_v.at[pl.ds(0, N)], sem),
           pltpu.async_copy(w_hbm, w_v, sem),
           pltpu.async_copy(b_hbm, b_v, sem)]
    for cp in cps:
        cp.wait()

    lanes = lax.iota(jnp.int32, L)

    def splat_total(vec):
        tmp_v[...] = plsc.cumsum(vec)
        return plsc.load_gather(tmp_v, [_splat(L - 1)])

    cidx = []
    for cc in range(NCH):
        raw = chars_v[pl.ds(cc * L, L)]
        if (cc + 1) * L > N:
            raw = jnp.where(lanes + cc * L < N, raw, 0)
        cidx.append(raw)

    vidx = [jnp.minimum(lanes + c * L, VOCAB - 1) for c in range(VCH)]
    b0 = plsc.load_gather(b_v, [_splat(js[0])])
    b1 = plsc.load_gather(b_v, [_splat(js[1])])
    acc = [[b0 for _ in range(VCH)], [b1 for _ in range(VCH)]]
    for k in range(EMB):
        w0 = plsc.load_gather(w_v, [_splat(js[0]), _splat(k)])
        w1 = plsc.load_gather(w_v, [_splat(js[1]), _splat(k)])
        for c in range(VCH):
            t = jnp.maximum(
                plsc.load_gather(table_v, [vidx[c], _splat(k)]), 0.0)
            acc[0][c] = acc[0][c] + w0 * t
            acc[1][c] = acc[1][c] + w1 * t

    def seg_sum(chunks, lo, hi):
        tot = None
        for cc in range(NCH):
            clo, chi = cc * L, (cc + 1) * L
            if chi <= lo or clo >= hi:
                continue
            x = chunks[cc]
            if clo < lo or chi > hi:
                rows = lanes + clo
                x = jnp.where((rows >= lo) & (rows < hi), x, 0.0)
            tot = x if tot is None else tot + x
        return splat_total(tot)

    def seg_var(chunks, lo, hi, mean):
        tot = None
        for cc in range(NCH):
            clo, chi = cc * L, (cc + 1) * L
            if chi <= lo or clo >= hi:
                continue
            d = chunks[cc] - mean
            d = d * d
            if clo < lo or chi > hi:
                rows = lanes + clo
                d = jnp.where((rows >= lo) & (rows < hi), d, 0.0)
            tot = d if tot is None else tot + d
        return splat_total(tot) * (1.0 / (hi - lo - 1))

    for jj in range(2):
        j = js[jj]
        for c in range(VCH):
            t2_v[pl.ds(c * L, L)] = acc[jj][c]

        chunks = []
        for cc in range(NCH):
            ch = plsc.load_gather(t2_v, [cidx[cc]])
            col_v[pl.ds(cc * L, L)] = ch
            chunks.append(ch)

        means = [seg_sum(chunks, lo, hi) * (1.0 / (hi - lo))
                 for lo, hi in _SEG_MEAN]
        nr, ar, sr = means
        d1, d2, d3 = nr - ar, sr - ar, nr - sr
        rd = d1 * d1 + d2 * d2 + d3 * d3

        std_means = [nr, seg_sum(chunks, 10, 26) * (1.0 / 16.0), sr]
        vs = [seg_var(chunks, lo, hi, m)
              for (lo, hi), m in zip(_SEG_STD, std_means)]
        v = jnp.where(lanes == 0, vs[0], jnp.where(lanes == 1, vs[1], vs[2]))
        x = (v + 1.0) * 0.5
        for _ in range(NEWTON_ITERS):
            x = 0.5 * (x + v / x)
        stds = jnp.where(lanes <= 2, x, 0.0)
        s_col = splat_total(stds)

        pltpu.sync_copy(col_v, embT_hbm.at[pl.ds(j * NPAD, NPAD)])
        part_v[...] = jnp.where(lanes == 0, s_col,
                                jnp.where(lanes == 1, rd, 0.0))
        pltpu.sync_copy(part_v, parts_hbm.at[pl.ds(j * L, L)])


@functools.lru_cache(maxsize=1)
def _build_sc_kernel():
  mesh = plsc.VectorSubcoreMesh(core_axis_name="c", subcore_axis_name="s",
                                num_cores=1)
  return functools.partial(
    pl.kernel,
    out_type=(
        jax.ShapeDtypeStruct((EMB * NPAD,), jnp.float32),
        jax.ShapeDtypeStruct((EMB * L,), jnp.float32),
    ),
    mesh=mesh,
    compiler_params=pltpu.CompilerParams(needs_layout_passes=False),
    scratch_types=[
        pltpu.VMEM((VOCAB, EMB), jnp.float32),
        pltpu.VMEM((NPAD,), jnp.int32),
        pltpu.VMEM((EMB, EMB), jnp.float32),
        pltpu.VMEM((EMB,), jnp.float32),
        pltpu.VMEM((VPAD,), jnp.float32),
        pltpu.VMEM((NPAD,), jnp.float32),
        pltpu.VMEM((L,), jnp.float32),
        pltpu.VMEM((L,), jnp.float32),
        pltpu.SemaphoreType.DMA,
    ],
  )(_sc_body)


def kernel(chars, table, W, b):
    embT, parts = _build_sc_kernel()(table, chars.astype(jnp.int32),
                                     W, b)
    emb = embT.reshape(EMB, NPAD)[:, :N].T
    p = parts.reshape(EMB, L)
    loss = p[:, 0].sum() + 1.0 / p[:, 1].sum()
    return (loss, emb)


# speedup vs baseline: 1.1746x; 1.1746x over previous
import functools
import jax
import jax.numpy as jnp
from jax import lax
from jax.experimental import pallas as pl
from jax.experimental.pallas import tpu as pltpu
from jax.experimental.pallas import tpu_sc as plsc

N = 68
NPAD = 80
VOCAB = 101
VPAD = 112
EMB = 32
L = 16
NCH = NPAD // L
VCH = VPAD // L
NEWTON_ITERS = 16

_SEG_MEAN = ((0, 10), (10, 36), (36, N))
_SEG_STD = ((0, 10), (10, 26), (36, N))


def _splat(val, dtype=jnp.int32):
    return jnp.full((L,), val, dtype)


def _sc_body(tT_hbm, chars_hbm, w_hbm, b_hbm, embT_hbm, parts_hbm, loss_hbm,
             tT_v, chars_v, w_v, b_v, t2_v, col_v, tmp_v, part_v, red_v,
             sem):
    sid = lax.axis_index("s")

    cps = [pltpu.async_copy(tT_hbm, tT_v, sem),
           pltpu.async_copy(chars_hbm, chars_v, sem),
           pltpu.async_copy(w_hbm, w_v, sem),
           pltpu.async_copy(b_hbm, b_v, sem)]
    for cp in cps:
        cp.wait()

    lanes = lax.iota(jnp.int32, L)

    def splat_total(vec):
        tmp_v[...] = plsc.cumsum(vec)
        return plsc.load_gather(tmp_v, [_splat(L - 1)])

    def seg_sum(chunks, lo, hi):
        tot = None
        for cc in range(NCH):
            clo, chi = cc * L, (cc + 1) * L
            if chi <= lo or clo >= hi:
                continue
            x = chunks[cc]
            if clo < lo or chi > hi:
                rows = lanes + clo
                x = jnp.where((rows >= lo) & (rows < hi), x, 0.0)
            tot = x if tot is None else tot + x
        return splat_total(tot)

    def seg_var(chunks, lo, hi, mean):
        tot = None
        for cc in range(NCH):
            clo, chi = cc * L, (cc + 1) * L
            if chi <= lo or clo >= hi:
                continue
            d = chunks[cc] - mean
            d = d * d
            if clo < lo or chi > hi:
                rows = lanes + clo
                d = jnp.where((rows >= lo) & (rows < hi), d, 0.0)
            tot = d if tot is None else tot + d
        return splat_total(tot) * (1.0 / (hi - lo - 1))

    for j in (sid, sid + L):
        bj = plsc.load_gather(b_v, [_splat(j)])
        acc = [bj for _ in range(VCH)]
        for k in range(EMB):
            wk = plsc.load_gather(w_v, [_splat(j), _splat(k)])
            for c in range(VCH):
                t = jnp.maximum(tT_v[k, pl.ds(c * L, L)], 0.0)
                acc[c] = acc[c] + wk * t
        for c in range(VCH):
            t2_v[pl.ds(c * L, L)] = acc[c]

        chunks = []
        for cc in range(NCH):
            ch = plsc.load_gather(t2_v, [chars_v[pl.ds(cc * L, L)]])
            col_v[pl.ds(cc * L, L)] = ch
            chunks.append(ch)

        means = [seg_sum(chunks, lo, hi) * (1.0 / (hi - lo))
                 for lo, hi in _SEG_MEAN]
        nr, ar, sr = means
        d1, d2, d3 = nr - ar, sr - ar, nr - sr
        rd = d1 * d1 + d2 * d2 + d3 * d3

        std_means = [nr, seg_sum(chunks, 10, 26) * (1.0 / 16.0), sr]
        vs = [seg_var(chunks, lo, hi, m)
              for (lo, hi), m in zip(_SEG_STD, std_means)]
        v = jnp.where(lanes == 0, vs[0], jnp.where(lanes == 1, vs[1], vs[2]))
        x = (v + 1.0) * 0.5
        for _ in range(NEWTON_ITERS):
            x = 0.5 * (x + v / x)
        stds = jnp.where(lanes <= 2, x, 0.0)
        s_col = splat_total(stds)

        pltpu.sync_copy(col_v, embT_hbm.at[pl.ds(j * NPAD, NPAD)])
        part_v[...] = jnp.where(lanes == 0, s_col,
                                jnp.where(lanes == 1, rd, 0.0))
        pltpu.sync_copy(part_v, parts_hbm.at[pl.ds(j * L, L)])

    plsc.subcore_barrier()

    @pl.when(sid == 0)
    def _():
        pltpu.sync_copy(parts_hbm, red_v)
        s0 = plsc.load_gather(red_v, [lanes * L])
        s1 = plsc.load_gather(red_v, [(lanes + L) * L])
        r0 = plsc.load_gather(red_v, [lanes * L + 1])
        r1 = plsc.load_gather(red_v, [(lanes + L) * L + 1])
        loss = splat_total(s0 + s1) + 1.0 / splat_total(r0 + r1)
        part_v[...] = jnp.where(lanes == 0, loss, 0.0)
        pltpu.sync_copy(part_v, loss_hbm)


@functools.lru_cache(maxsize=1)
def _build_sc_kernel():
  mesh = plsc.VectorSubcoreMesh(core_axis_name="c", subcore_axis_name="s",
                                num_cores=1)
  return functools.partial(
    pl.kernel,
    out_type=(
        jax.ShapeDtypeStruct((EMB * NPAD,), jnp.float32),
        jax.ShapeDtypeStruct((EMB * L,), jnp.float32),
        jax.ShapeDtypeStruct((L,), jnp.float32),
    ),
    mesh=mesh,
    compiler_params=pltpu.CompilerParams(needs_layout_passes=False),
    scratch_types=[
        pltpu.VMEM((EMB, VPAD), jnp.float32),
        pltpu.VMEM((NPAD,), jnp.int32),
        pltpu.VMEM((EMB, EMB), jnp.float32),
        pltpu.VMEM((EMB,), jnp.float32),
        pltpu.VMEM((VPAD,), jnp.float32),
        pltpu.VMEM((NPAD,), jnp.float32),
        pltpu.VMEM((L,), jnp.float32),
        pltpu.VMEM((L,), jnp.float32),
        pltpu.VMEM((EMB * L,), jnp.float32),
        pltpu.SemaphoreType.DMA,
    ],
  )(_sc_body)


def kernel(chars, table, W, b):
    chars_p = jnp.zeros((NPAD,), jnp.int32).at[:N].set(chars.astype(jnp.int32))
    tT = jnp.zeros((EMB, VPAD), jnp.float32).at[:, :VOCAB].set(table.T)
    embT, _parts, lossv = _build_sc_kernel()(tT, chars_p,
                                             W.astype(jnp.float32),
                                             b.astype(jnp.float32))
    emb = embT.reshape(EMB, NPAD)[:, :N].T
    return (lossv[0], emb)
